# baseline (speedup 1.0000x reference)
import functools

import jax
import jax.numpy as jnp
from jax import lax
from jax.experimental import pallas as pl
from jax.experimental.pallas import tpu as pltpu
from jax.experimental.pallas import tpu_sc as plsc

B = 8
T_PHN = 512
H = 256
MEL = 2048
LANES = 16

NW = 32
ROWS_PER_W = (B * MEL) // NW
CHUNK = 64
NBUF = 4
NCHUNK = ROWS_PER_W // CHUNK
ZROW = B * T_PHN
W_PER_B = MEL // ROWS_PER_W


def _lr_body(tbl, dur, out, mel, dur_v, idx2, meldur_v, mel_v,
             b0, b1, b2, b3, g0, g1, g2, g3, s0, s1, s2, s3, msem):
    bufs = (b0, b1, b2, b3)
    gsems = (g0, g1, g2, g3)
    ssems = (s0, s1, s2, s3)

    cid = lax.axis_index("c")
    sid = lax.axis_index("s")
    wid = sid * 2 + cid
    base = wid * ROWS_PER_W
    b = wid // W_PER_B

    pltpu.sync_copy(dur.at[pl.ds(base, ROWS_PER_W)], dur_v)
    off = b * T_PHN - 1
    for c in range(NCHUNK):
        row = idx2.at[c]
        for j in range(CHUNK // LANES):
            d = dur_v[pl.ds(c * CHUNK + j * LANES, LANES)]
            row[pl.ds(j * LANES, LANES)] = jnp.where(d == 0, ZROW, d + off)

    gh = [None] * NCHUNK
    sh = [None] * NCHUNK
    for c in range(NBUF):
        gh[c] = pltpu.async_copy(tbl.at[idx2.at[c]], bufs[c], gsems[c])

    @pl.when(wid < B)
    def _mel():
        pltpu.sync_copy(dur.at[pl.ds(wid * MEL, MEL)], meldur_v)
        lanes = lax.iota(jnp.int32, LANES)

        def scan_body(i, acc):
            dvec = meldur_v[pl.ds(i * LANES, LANES)]
            return jnp.minimum(acc, jnp.where(dvec == 0, lanes + i * LANES, MEL))

        acc = lax.fori_loop(0, MEL // LANES, scan_body,
                            jnp.full((LANES,), MEL, jnp.int32))
        dnums = lax.GatherDimensionNumbers(
            offset_dims=(), collapsed_slice_dims=(0,), start_index_map=(0,))
        for s in (8, 4, 2, 1):
            perm = jnp.bitwise_xor(lanes, s)
            shuf = lax.gather(acc, perm[:, None], dnums, slice_sizes=(1,),
                              mode=lax.GatherScatterMode.PROMISE_IN_BOUNDS)
            acc = jnp.minimum(acc, shuf)
        mel_v[...] = acc
        widx = jnp.full((LANES,), wid, jnp.int32)
        pltpu.async_copy(mel_v, mel.at[widx], msem).wait()

    for c in range(NCHUNK):
        slot = c % NBUF
        gh[c].wait()
        sh[c] = pltpu.async_copy(
            bufs[slot], out.at[pl.ds(base + c * CHUNK, CHUNK)], ssems[slot])
        nxt = c + NBUF
        if nxt < NCHUNK:
            sh[c].wait()
            gh[nxt] = pltpu.async_copy(tbl.at[idx2.at[nxt]], bufs[slot], gsems[slot])
    for c in range(NCHUNK - NBUF, NCHUNK):
        sh[c].wait()


_lr_call = pl.kernel(
    _lr_body,
    out_type=(
        jax.ShapeDtypeStruct((B * MEL, H), jnp.float32),
        jax.ShapeDtypeStruct((B,), jnp.int32),
    ),
    mesh=plsc.VectorSubcoreMesh(core_axis_name="c", subcore_axis_name="s"),
    scratch_types=(
        pltpu.VMEM((ROWS_PER_W,), jnp.int32),
        pltpu.VMEM((NCHUNK, CHUNK), jnp.int32),
        pltpu.VMEM((MEL,), jnp.int32),
        pltpu.VMEM((LANES,), jnp.int32),
        pltpu.VMEM((CHUNK, H), jnp.float32),
        pltpu.VMEM((CHUNK, H), jnp.float32),
        pltpu.VMEM((CHUNK, H), jnp.float32),
        pltpu.VMEM((CHUNK, H), jnp.float32),
        pltpu.SemaphoreType.DMA,
        pltpu.SemaphoreType.DMA,
        pltpu.SemaphoreType.DMA,
        pltpu.SemaphoreType.DMA,
        pltpu.SemaphoreType.DMA,
        pltpu.SemaphoreType.DMA,
        pltpu.SemaphoreType.DMA,
        pltpu.SemaphoreType.DMA,
        pltpu.SemaphoreType.DMA,
    ),
)


def kernel(x, duration):
    dur = duration.astype(jnp.int32).reshape(B * MEL)
    tbl = jnp.concatenate(
        [x.reshape(B * T_PHN, H), jnp.zeros((8, H), jnp.float32)], axis=0)
    out_flat, mel_len = _lr_call(tbl, dur)
    return out_flat.reshape(B, MEL, H), mel_len


# speedup vs baseline: 1.0046x; 1.0046x over previous
import functools

import jax
import jax.numpy as jnp
from jax import lax
from jax.experimental import pallas as pl
from jax.experimental.pallas import tpu as pltpu
from jax.experimental.pallas import tpu_sc as plsc

B = 8
T_PHN = 512
H = 256
MEL = 2048
LANES = 16

NW = 32
ROWS_PER_W = (B * MEL) // NW
CHUNK = 64
NBUF = 4
NCHUNK = ROWS_PER_W // CHUNK
ZROW = B * T_PHN
W_PER_B = MEL // ROWS_PER_W


def _lr_body(tbl, dur, out, mel, dur_v, idx2, meldur_v, mel_v,
             b0, b1, b2, b3, g0, g1, g2, g3, s0, s1, s2, s3, msem):
    bufs = (b0, b1, b2, b3)
    gsems = (g0, g1, g2, g3)
    ssems = (s0, s1, s2, s3)

    cid = lax.axis_index("c")
    sid = lax.axis_index("s")
    wid = sid * 2 + cid
    base = wid * ROWS_PER_W
    b = wid // W_PER_B

    pltpu.sync_copy(dur.at[pl.ds(base, ROWS_PER_W)], dur_v)
    off = b * T_PHN - 1
    for c in range(NCHUNK):
        row = idx2.at[c]
        for j in range(CHUNK // LANES):
            d = dur_v[pl.ds(c * CHUNK + j * LANES, LANES)]
            row[pl.ds(j * LANES, LANES)] = jnp.where(d == 0, ZROW, d + off)

    gh = [None] * NCHUNK
    sh = [None] * NCHUNK
    for c in range(NBUF):
        gh[c] = pltpu.async_copy(tbl.at[idx2.at[c]], bufs[c], gsems[c])

    @pl.when(wid < B)
    def _mel():
        pltpu.sync_copy(dur.at[pl.ds(wid * MEL, MEL)], meldur_v)
        lanes = lax.iota(jnp.int32, LANES)

        acc = jnp.full((LANES,), MEL, jnp.int32)
        for i in range(MEL // LANES):
            dvec = meldur_v[pl.ds(i * LANES, LANES)]
            acc = jnp.minimum(acc, jnp.where(dvec == 0, lanes + i * LANES, MEL))
        dnums = lax.GatherDimensionNumbers(
            offset_dims=(), collapsed_slice_dims=(0,), start_index_map=(0,))
        for s in (8, 4, 2, 1):
            perm = jnp.bitwise_xor(lanes, s)
            shuf = lax.gather(acc, perm[:, None], dnums, slice_sizes=(1,),
                              mode=lax.GatherScatterMode.PROMISE_IN_BOUNDS)
            acc = jnp.minimum(acc, shuf)
        mel_v[...] = acc
        widx = jnp.full((LANES,), wid, jnp.int32)
        pltpu.async_copy(mel_v, mel.at[widx], msem).wait()

    for c in range(NCHUNK):
        slot = c % NBUF
        gh[c].wait()
        sh[c] = pltpu.async_copy(
            bufs[slot], out.at[pl.ds(base + c * CHUNK, CHUNK)], ssems[slot])
        nxt = c + NBUF
        if nxt < NCHUNK:
            sh[c].wait()
            gh[nxt] = pltpu.async_copy(tbl.at[idx2.at[nxt]], bufs[slot], gsems[slot])
    for c in range(NCHUNK - NBUF, NCHUNK):
        sh[c].wait()


_lr_call = pl.kernel(
    _lr_body,
    out_type=(
        jax.ShapeDtypeStruct((B * MEL, H), jnp.float32),
        jax.ShapeDtypeStruct((B,), jnp.int32),
    ),
    mesh=plsc.VectorSubcoreMesh(core_axis_name="c", subcore_axis_name="s"),
    scratch_types=(
        pltpu.VMEM((ROWS_PER_W,), jnp.int32),
        pltpu.VMEM((NCHUNK, CHUNK), jnp.int32),
        pltpu.VMEM((MEL,), jnp.int32),
        pltpu.VMEM((LANES,), jnp.int32),
        pltpu.VMEM((CHUNK, H), jnp.float32),
        pltpu.VMEM((CHUNK, H), jnp.float32),
        pltpu.VMEM((CHUNK, H), jnp.float32),
        pltpu.VMEM((CHUNK, H), jnp.float32),
        pltpu.SemaphoreType.DMA,
        pltpu.SemaphoreType.DMA,
        pltpu.SemaphoreType.DMA,
        pltpu.SemaphoreType.DMA,
        pltpu.SemaphoreType.DMA,
        pltpu.SemaphoreType.DMA,
        pltpu.SemaphoreType.DMA,
        pltpu.SemaphoreType.DMA,
        pltpu.SemaphoreType.DMA,
    ),
)


def kernel(x, duration):
    dur = duration.astype(jnp.int32).reshape(B * MEL)
    tbl = jnp.concatenate(
        [x.reshape(B * T_PHN, H), jnp.zeros((8, H), jnp.float32)], axis=0)
    out_flat, mel_len = _lr_call(tbl, dur)
    return out_flat.reshape(B, MEL, H), mel_len


# speedup vs baseline: 1.3737x; 1.3675x over previous
import functools

import jax
import jax.numpy as jnp
from jax import lax
from jax.experimental import pallas as pl
from jax.experimental.pallas import tpu as pltpu
from jax.experimental.pallas import tpu_sc as plsc

B = 8
T_PHN = 512
H = 256
MEL = 2048
LANES = 16

NW = 32
ROWS_PER_W = (B * MEL) // NW
CHUNK = 64
NBUF = 4
NCHUNK = ROWS_PER_W // CHUNK
ZROW = B * T_PHN
W_PER_B = MEL // ROWS_PER_W


def _lr_body(tbl, dur, out, mel, dur_v, idx2, meldur_v, mel_v,
             b0, b1, b2, b3, g0, g1, g2, g3, s0, s1, s2, s3, msem):
    bufs = (b0, b1, b2, b3)
    gsems = (g0, g1, g2, g3)
    ssems = (s0, s1, s2, s3)

    cid = lax.axis_index("c")
    sid = lax.axis_index("s")
    wid = sid * 2 + cid
    base = wid * ROWS_PER_W
    b = wid // W_PER_B

    pltpu.sync_copy(dur.at[pl.ds(base, ROWS_PER_W)], dur_v)
    off = b * T_PHN - 1
    for c in range(NCHUNK):
        row = idx2.at[c]
        for j in range(CHUNK // LANES):
            d = dur_v[pl.ds(c * CHUNK + j * LANES, LANES)]
            row[pl.ds(j * LANES, LANES)] = jnp.where(d == 0, ZROW, d + off)

    gh = [None] * NCHUNK
    sh = [None] * NCHUNK
    for c in range(NBUF):
        gh[c] = pltpu.async_copy(tbl.at[idx2.at[c]], bufs[c], gsems[c])

    @pl.when(wid < B)
    def _mel():
        pltpu.sync_copy(dur.at[pl.ds(wid * MEL, MEL)], meldur_v)
        lanes = lax.iota(jnp.int32, LANES)

        def scan_body(i, acc):
            dvec = meldur_v[pl.ds(i * LANES, LANES)]
            return jnp.minimum(acc, jnp.where(dvec == 0, lanes + i * LANES, MEL))

        acc = lax.fori_loop(0, MEL // LANES, scan_body,
                            jnp.full((LANES,), MEL, jnp.int32))
        dnums = lax.GatherDimensionNumbers(
            offset_dims=(), collapsed_slice_dims=(0,), start_index_map=(0,))
        for s in (8, 4, 2, 1):
            perm = jnp.bitwise_xor(lanes, s)
            shuf = lax.gather(acc, perm[:, None], dnums, slice_sizes=(1,),
                              mode=lax.GatherScatterMode.PROMISE_IN_BOUNDS)
            acc = jnp.minimum(acc, shuf)
        mel_v[...] = acc
        pltpu.async_copy(mel_v, mel.at[wid], msem).wait()

    for c in range(NCHUNK):
        slot = c % NBUF
        gh[c].wait()
        sh[c] = pltpu.async_copy(
            bufs[slot], out.at[pl.ds(base + c * CHUNK, CHUNK)], ssems[slot])
        nxt = c + NBUF
        if nxt < NCHUNK:
            sh[c].wait()
            gh[nxt] = pltpu.async_copy(tbl.at[idx2.at[nxt]], bufs[slot], gsems[slot])
    for c in range(NCHUNK - NBUF, NCHUNK):
        sh[c].wait()


_lr_call = pl.kernel(
    _lr_body,
    out_type=(
        jax.ShapeDtypeStruct((B * MEL, H), jnp.float32),
        jax.ShapeDtypeStruct((B, LANES), jnp.int32),
    ),
    mesh=plsc.VectorSubcoreMesh(core_axis_name="c", subcore_axis_name="s"),
    scratch_types=(
        pltpu.VMEM((ROWS_PER_W,), jnp.int32),
        pltpu.VMEM((NCHUNK, CHUNK), jnp.int32),
        pltpu.VMEM((MEL,), jnp.int32),
        pltpu.VMEM((LANES,), jnp.int32),
        pltpu.VMEM((CHUNK, H), jnp.float32),
        pltpu.VMEM((CHUNK, H), jnp.float32),
        pltpu.VMEM((CHUNK, H), jnp.float32),
        pltpu.VMEM((CHUNK, H), jnp.float32),
        pltpu.SemaphoreType.DMA,
        pltpu.SemaphoreType.DMA,
        pltpu.SemaphoreType.DMA,
        pltpu.SemaphoreType.DMA,
        pltpu.SemaphoreType.DMA,
        pltpu.SemaphoreType.DMA,
        pltpu.SemaphoreType.DMA,
        pltpu.SemaphoreType.DMA,
        pltpu.SemaphoreType.DMA,
    ),
)


def kernel(x, duration):
    dur = duration.astype(jnp.int32).reshape(B * MEL)
    tbl = jnp.concatenate(
        [x.reshape(B * T_PHN, H), jnp.zeros((8, H), jnp.float32)], axis=0)
    out_flat, mel_rows = _lr_call(tbl, dur)
    return out_flat.reshape(B, MEL, H), mel_rows[:, 0]


# speedup vs baseline: 1.4053x; 1.0230x over previous
import functools

import jax
import jax.numpy as jnp
from jax import lax
from jax.experimental import pallas as pl
from jax.experimental.pallas import tpu as pltpu
from jax.experimental.pallas import tpu_sc as plsc

B = 8
T_PHN = 512
H = 256
MEL = 2048
LANES = 16

NW = 32
ROWS_PER_W = (B * MEL) // NW
CHUNK = 64
NBUF = 4
NCHUNK = ROWS_PER_W // CHUNK
ZROW = B * T_PHN
W_PER_B = MEL // ROWS_PER_W


def _lr_body(x, dur, out, mel, dur_v, idx2, meldur_v, mel_v,
             b0, b1, b2, b3, g0, g1, g2, g3, s0, s1, s2, s3, msem):
    bufs = (b0, b1, b2, b3)
    gsems = (g0, g1, g2, g3)
    ssems = (s0, s1, s2, s3)

    cid = lax.axis_index("c")
    sid = lax.axis_index("s")
    wid = sid * 2 + cid
    base = wid * ROWS_PER_W
    b = wid // W_PER_B

    pltpu.sync_copy(dur.at[pl.ds(base, ROWS_PER_W)], dur_v.at[pl.ds(0, ROWS_PER_W)])
    off = b * T_PHN
    zero16 = jnp.zeros((LANES,), jnp.int32)
    lanes16 = lax.iota(jnp.int32, LANES)
    gdnums = lax.GatherDimensionNumbers(
        offset_dims=(), collapsed_slice_dims=(0,), start_index_map=(0,))

    def _lane_min(v):
        for s in (8, 4, 2, 1):
            perm = jnp.bitwise_xor(lanes16, s)
            shuf = lax.gather(v, perm[:, None], gdnums, slice_sizes=(1,),
                              mode=lax.GatherScatterMode.PROMISE_IN_BOUNDS)
            v = jnp.minimum(v, shuf)
        return v

    zany = []
    for c in range(NCHUNK):
        row = idx2.at[c]
        dmin = jnp.full((LANES,), 1, jnp.int32)
        for j in range(CHUNK // LANES):
            d = dur_v[pl.ds(c * CHUNK + j * LANES, LANES)]
            row[pl.ds(j * LANES, LANES)] = jnp.maximum(d - 1, zero16) + off
            dmin = jnp.minimum(dmin, d)
        zany.append(_lane_min(dmin)[0] == 0)

    gh = [None] * NCHUNK
    sh = [None] * NCHUNK
    for c in range(NBUF):
        gh[c] = pltpu.async_copy(x.at[idx2.at[c]], bufs[c], gsems[c])

    @pl.when(wid < B)
    def _mel():
        pltpu.sync_copy(dur.at[pl.ds(wid * MEL, MEL)], meldur_v)
        lanes = lax.iota(jnp.int32, LANES)

        def scan_body(i, acc):
            dvec = meldur_v[pl.ds(i * LANES, LANES)]
            return jnp.minimum(acc, jnp.where(dvec == 0, lanes + i * LANES, MEL))

        acc = lax.fori_loop(0, MEL // LANES, scan_body,
                            jnp.full((LANES,), MEL, jnp.int32))
        mel_v[...] = _lane_min(acc)
        pltpu.async_copy(mel_v, mel.at[wid], msem).wait()

    zrow = jnp.zeros((LANES,), jnp.float32)

    for c in range(NCHUNK):
        slot = c % NBUF
        gh[c].wait()

        @pl.when(zany[c])
        def _fix(c=c, slot=slot):
            def fix_body(r, carry):
                dvec = dur_v[pl.ds(c * CHUNK + r, LANES)]

                @pl.when(dvec[0] == 0)
                def _z():
                    brow = bufs[slot].at[r]
                    for k in range(H // LANES):
                        brow[pl.ds(k * LANES, LANES)] = zrow
                return carry

            lax.fori_loop(0, CHUNK, fix_body, 0)

        sh[c] = pltpu.async_copy(
            bufs[slot], out.at[pl.ds(base + c * CHUNK, CHUNK)], ssems[slot])
        nxt = c + NBUF
        if nxt < NCHUNK:
            sh[c].wait()
            gh[nxt] = pltpu.async_copy(x.at[idx2.at[nxt]], bufs[slot], gsems[slot])
    for c in range(NCHUNK - NBUF, NCHUNK):
        sh[c].wait()


_lr_call = pl.kernel(
    _lr_body,
    out_type=(
        jax.ShapeDtypeStruct((B * MEL, H), jnp.float32),
        jax.ShapeDtypeStruct((B, LANES), jnp.int32),
    ),
    mesh=plsc.VectorSubcoreMesh(core_axis_name="c", subcore_axis_name="s"),
    scratch_types=(
        pltpu.VMEM((ROWS_PER_W + LANES,), jnp.int32),
        pltpu.VMEM((NCHUNK, CHUNK), jnp.int32),
        pltpu.VMEM((MEL,), jnp.int32),
        pltpu.VMEM((LANES,), jnp.int32),
        pltpu.VMEM((CHUNK, H), jnp.float32),
        pltpu.VMEM((CHUNK, H), jnp.float32),
        pltpu.VMEM((CHUNK, H), jnp.float32),
        pltpu.VMEM((CHUNK, H), jnp.float32),
        pltpu.SemaphoreType.DMA,
        pltpu.SemaphoreType.DMA,
        pltpu.SemaphoreType.DMA,
        pltpu.SemaphoreType.DMA,
        pltpu.SemaphoreType.DMA,
        pltpu.SemaphoreType.DMA,
        pltpu.SemaphoreType.DMA,
        pltpu.SemaphoreType.DMA,
        pltpu.SemaphoreType.DMA,
    ),
)


def kernel(x, duration):
    dur = duration.astype(jnp.int32).reshape(B * MEL)
    out_flat, mel_rows = _lr_call(x.reshape(B * T_PHN, H), dur)
    return out_flat.reshape(B, MEL, H), mel_rows[:, 0]
